# baseline (speedup 1.0000x reference)
import jax
import jax.numpy as jnp
from jax.experimental import pallas as pl

NT = 64


def _pool_body(x_ref, o_ref):
    win = x_ref.shape[2]
    o_ref[0, :, :] = jnp.sum(x_ref[0], axis=1) * (1.0 / win)


def kernel(features):
    B, N, D = features.shape
    win = N // NT
    group = 8
    nt_blk = NT * group
    x = features.reshape(B // group, nt_blk, win, D)
    out = pl.pallas_call(
        _pool_body,
        grid=(B // group,),
        in_specs=[pl.BlockSpec((1, nt_blk, win, D), lambda b: (b, 0, 0, 0))],
        out_specs=pl.BlockSpec((1, nt_blk, D), lambda b: (b, 0, 0)),
        out_shape=jax.ShapeDtypeStruct((B // group, nt_blk, D), jnp.float32),
    )(x)
    return out.reshape(B, NT, D)


# speedup vs baseline: 1.0681x; 1.0681x over previous
import jax
import jax.numpy as jnp
from jax.experimental import pallas as pl
from jax.experimental.pallas import tpu as pltpu

NT = 64
WIN = 16

_NBUF = 8
_ROWS = 64


def _pool_body(x_hbm, o_ref, buf, sems):
    i = pl.program_id(0)
    nsteps = pl.num_programs(0)

    def start(block, slot):
        pltpu.make_async_copy(
            x_hbm.at[pl.ds(block * _ROWS, _ROWS)],
            buf.at[slot],
            sems.at[slot],
        ).start()

    @pl.when(i == 0)
    def _():
        for k in range(_NBUF):
            start(k, k)

    slot = lax.rem(i, _NBUF)
    pltpu.make_async_copy(
        x_hbm.at[pl.ds(0, _ROWS)],
        buf.at[slot],
        sems.at[slot],
    ).wait()

    o_ref[...] = jnp.sum(buf[slot], axis=1) * (1.0 / WIN)

    @pl.when(i + _NBUF < nsteps)
    def _():
        start(i + _NBUF, slot)


from jax import lax


def kernel(features):
    B, N, D = features.shape
    nblocks = B * N // (_ROWS * WIN)
    x = features.reshape(B * N // WIN, WIN, D)
    out = pl.pallas_call(
        _pool_body,
        grid=(nblocks,),
        in_specs=[pl.BlockSpec(memory_space=pl.ANY)],
        out_specs=pl.BlockSpec((_ROWS, D), lambda b: (b, 0)),
        out_shape=jax.ShapeDtypeStruct((B * NT, D), jnp.float32),
        scratch_shapes=[
            pltpu.VMEM((_NBUF, _ROWS, WIN, D), jnp.float32),
            pltpu.SemaphoreType.DMA((_NBUF,)),
        ],
    )(x)
    return out.reshape(B, NT, D)


# speedup vs baseline: 1.0868x; 1.0175x over previous
import jax
import jax.numpy as jnp
from jax.experimental import pallas as pl
from jax.experimental.pallas import tpu as pltpu

NT = 64
WIN = 16

_NBUF = 16
_ROWS = 64


def _pool_body(x_hbm, o_ref, buf, sems):
    i = pl.program_id(0)
    nsteps = pl.num_programs(0)

    def start(block, slot):
        pltpu.make_async_copy(
            x_hbm.at[pl.ds(block * _ROWS, _ROWS)],
            buf.at[slot],
            sems.at[slot],
        ).start()

    @pl.when(i == 0)
    def _():
        for k in range(_NBUF):
            start(k, k)

    slot = lax.rem(i, _NBUF)
    pltpu.make_async_copy(
        x_hbm.at[pl.ds(0, _ROWS)],
        buf.at[slot],
        sems.at[slot],
    ).wait()

    o_ref[...] = jnp.sum(buf[slot], axis=1) * (1.0 / WIN)

    @pl.when(i + _NBUF < nsteps)
    def _():
        start(i + _NBUF, slot)


from jax import lax


def kernel(features):
    B, N, D = features.shape
    nblocks = B * N // (_ROWS * WIN)
    x = features.reshape(B * N // WIN, WIN, D)
    out = pl.pallas_call(
        _pool_body,
        grid=(nblocks,),
        in_specs=[pl.BlockSpec(memory_space=pl.ANY)],
        out_specs=pl.BlockSpec((_ROWS, D), lambda b: (b, 0)),
        out_shape=jax.ShapeDtypeStruct((B * NT, D), jnp.float32),
        scratch_shapes=[
            pltpu.VMEM((_NBUF, _ROWS, WIN, D), jnp.float32),
            pltpu.SemaphoreType.DMA((_NBUF,)),
        ],
    )(x)
    return out.reshape(B, NT, D)
